# baseline (speedup 1.0000x reference)
import functools

import jax
import jax.numpy as jnp
from jax import lax
from jax.experimental import pallas as pl
from jax.experimental.pallas import tpu as pltpu
from jax.experimental.pallas import tpu_sc as plsc

NC = 2
NS = 16
LANES = 16
GRP = 256
NBUF = 4


def _hall_body(wc_ref, w_ref, x_ref, lo_ref, hi_ref):
    r = pl.program_id(1)
    nb = w_ref.shape[0]
    w_r = wc_ref[r, 0] * w_ref[0]
    for b in range(1, nb):
        w_r = w_r + wc_ref[r, b] * w_ref[b]
    h = jnp.dot(x_ref[...], w_r, preferred_element_type=jnp.float32)
    hf = h.shape[1] // 2
    lo_ref[0] = h[:, :hf]
    hi_ref[0] = h[:, hf:]


def _compute_h_halves(x, weight, w_comp):
    n, in_feat = x.shape
    nbases, _, out_feat = weight.shape
    nrels = w_comp.shape[0]
    half = out_feat // 2
    bn = 2000
    grid = (n // bn, nrels)
    return pl.pallas_call(
        _hall_body,
        grid=grid,
        in_specs=[
            pl.BlockSpec(memory_space=pltpu.SMEM),
            pl.BlockSpec((nbases, in_feat, out_feat), lambda i, r: (0, 0, 0)),
            pl.BlockSpec((bn, in_feat), lambda i, r: (i, 0)),
        ],
        out_specs=[
            pl.BlockSpec((1, bn, half), lambda i, r: (r, i, 0)),
            pl.BlockSpec((1, bn, half), lambda i, r: (r, i, 0)),
        ],
        out_shape=[
            jax.ShapeDtypeStruct((nrels, n, half), jnp.float32),
            jax.ShapeDtypeStruct((nrels, n, half), jnp.float32),
        ],
    )(w_comp, weight, x)


def _sc_body(n_nodes, half, gt, h_lo, h_hi, fia, dsta, norma, biast, out,
             acc, fib0, fib1, fib2, fib3, dstg0, dstg1, dstg2, dstg3,
             normg0, normg1, normg2, normg3, rows0, rows1, rows2, rows3,
             semi0, semi1, semi2, semi3, semg0, semg1, semg2, semg3,
             sems0, sems1, sems2, sems3):
    cc = lax.axis_index("c")
    ss = lax.axis_index("s")
    fibs = (fib0, fib1, fib2, fib3)
    dstgs = (dstg0, dstg1, dstg2, dstg3)
    normgs = (normg0, normg1, normg2, normg3)
    rowss = (rows0, rows1, rows2, rows3)
    semis = (semi0, semi1, semi2, semi3)
    semgs = (semg0, semg1, semg2, semg3)
    semss = (sems0, sems1, sems2, sems3)

    rows_per_tile = n_nodes // NS
    base = ss * rows_per_tile
    pltpu.sync_copy(biast.at[cc], rows0)
    full, rem = divmod(rows_per_tile, GRP)
    for k in range(full):
        pltpu.sync_copy(rows0, acc.at[pl.ds(base + k * GRP, GRP)])
    if rem:
        pltpu.sync_copy(rows0.at[pl.ds(0, rem)],
                        acc.at[pl.ds(base + full * GRP, rem)])
    plsc.subcore_barrier()

    tile_e0 = ss * (gt * GRP)

    def start_idx(g, b):
        e0 = tile_e0 + g * GRP
        pltpu.async_copy(fia.at[pl.ds(e0, GRP)], fibs[b], semis[b])
        pltpu.async_copy(dsta.at[pl.ds(e0, GRP)], dstgs[b], semis[b])
        pltpu.async_copy(norma.at[pl.ds(e0, GRP)], normgs[b], semis[b])

    def wait_idx(g, b):
        e0 = tile_e0 + g * GRP
        pltpu.make_async_copy(fia.at[pl.ds(e0, GRP)], fibs[b],
                              semis[b]).wait()
        pltpu.make_async_copy(dsta.at[pl.ds(e0, GRP)], dstgs[b],
                              semis[b]).wait()
        pltpu.make_async_copy(norma.at[pl.ds(e0, GRP)], normgs[b],
                              semis[b]).wait()

    def start_gather(b):
        @pl.when(cc == 0)
        def _():
            pltpu.async_copy(h_lo.at[fibs[b]], rowss[b], semgs[b])

        @pl.when(cc == 1)
        def _():
            pltpu.async_copy(h_hi.at[fibs[b]], rowss[b], semgs[b])

    def wait_gather(b):
        @pl.when(cc == 0)
        def _():
            pltpu.make_async_copy(h_lo.at[fibs[b]], rowss[b],
                                  semgs[b]).wait()

        @pl.when(cc == 1)
        def _():
            pltpu.make_async_copy(h_hi.at[fibs[b]], rowss[b],
                                  semgs[b]).wait()

    def scale(b):
        rows = rowss[b]
        normg = normgs[b]

        @plsc.parallel_loop(0, GRP // LANES, unroll=2)
        def _(q):
            nv = normg[pl.ds(q * LANES, LANES)]
            i0 = q * LANES
            for l in range(LANES):
                nbc = lax.gather(
                    nv, jnp.full((LANES, 1), l, jnp.int32),
                    lax.GatherDimensionNumbers(
                        offset_dims=(), collapsed_slice_dims=(0,),
                        start_index_map=(0,)),
                    (1,), mode=lax.GatherScatterMode.PROMISE_IN_BOUNDS)
                for j in range(half // LANES):
                    v = rows[i0 + l, pl.ds(j * LANES, LANES)]
                    rows[i0 + l, pl.ds(j * LANES, LANES)] = v * nbc

    def start_scatter(b):
        pltpu.async_copy(rowss[b], acc.at[dstgs[b]], semss[b], add=True)

    def wait_scatter(b):
        pltpu.make_async_copy(rowss[b], acc.at[dstgs[b]], semss[b]).wait()

    start_idx(0, 0)
    start_idx(1, 1)
    wait_idx(0, 0)
    start_gather(0)
    steady = gt - 1

    def body(i, b, static):
        bn1 = (b + 1) % NBUF
        bn2 = (b + 2) % NBUF
        if static:
            if i >= 2:
                wait_scatter(bn2)
            if i < gt - 2:
                start_idx(i + 2, bn2)
        else:
            @pl.when(i >= 2)
            def _():
                wait_scatter(bn2)

            @pl.when(i < gt - 2)
            def _():
                start_idx(i + 2, bn2)
        wait_idx(i + 1, bn1)
        start_gather(bn1)
        wait_gather(b)
        scale(b)
        start_scatter(b)

    def block(i0, _):
        for k in range(NBUF):
            body(i0 + k, k, False)
        return 0

    lax.fori_loop(0, steady // NBUF, lambda t, c: block(t * NBUF, c), 0)
    for i in range(steady - steady % NBUF, steady):
        body(i, i % NBUF, True)
    bl = (gt - 1) % NBUF
    wait_gather(bl)
    scale(bl)
    start_scatter(bl)
    wait_scatter((bl + 2) % NBUF)
    wait_scatter((bl + 3) % NBUF)
    wait_scatter(bl)
    plsc.subcore_barrier()

    pltpu.sync_copy(acc.at[pl.ds(base, rows_per_tile)],
                    out.at[pl.ds(base, rows_per_tile),
                           pl.ds(cc * half, half)])


def kernel(x, edge_index, etypes, norm, weight, w_comp, h_bias):
    n, in_feat = x.shape
    out_feat = weight.shape[2]
    nrels = w_comp.shape[0]
    e = etypes.shape[0]
    half = out_feat // 2

    h_lo, h_hi = _compute_h_halves(x, weight, w_comp)
    h_lo = h_lo.reshape(nrels * n, half)
    h_hi = h_hi.reshape(nrels * n, half)

    chunk = NS * GRP
    e_pad = ((e + chunk - 1) // chunk) * chunk
    pad = e_pad - e
    pad_iota = jnp.arange(pad, dtype=jnp.int32) % n
    fia = jnp.concatenate(
        [etypes.astype(jnp.int32) * n + edge_index[0].astype(jnp.int32),
         pad_iota])
    dsta = jnp.concatenate([edge_index[1].astype(jnp.int32), pad_iota])
    norma = jnp.concatenate([norm.reshape(e).astype(jnp.float32),
                             jnp.zeros((pad,), jnp.float32)])
    biast = jnp.broadcast_to(h_bias.reshape(2, 1, half), (2, GRP, half))
    gt = e_pad // (NS * GRP)

    mesh = plsc.VectorSubcoreMesh(core_axis_name="c", subcore_axis_name="s")
    sc_call = functools.partial(
        pl.kernel,
        out_type=jax.ShapeDtypeStruct((n, out_feat), jnp.float32),
        mesh=mesh,
        scratch_types=(
            [pltpu.VMEM_SHARED((n, half), jnp.float32)]
            + [pltpu.VMEM((GRP,), jnp.int32)] * 8
            + [pltpu.VMEM((GRP,), jnp.float32)] * 4
            + [pltpu.VMEM((GRP, half), jnp.float32)] * 4
            + [pltpu.SemaphoreType.DMA] * 12
        ),
        compiler_params=pltpu.CompilerParams(use_tc_tiling_on_sc=False),
    )(functools.partial(_sc_body, n, half, gt))
    return sc_call(h_lo, h_hi, fia, dsta, norma, biast)
